# Initial kernel scaffold; baseline (speedup 1.0000x reference)
import jax
import jax.numpy as jnp
from jax.experimental import pallas as pl


def kernel(x, edge_index, W_in, b_in, Wl, bl, Wr, gamma, beta, W1, b1, W2, b2):
    raise NotImplementedError("write your pallas kernel here")


# speedup vs baseline: 4.1138x; 4.1138x over previous
import functools

import jax
import jax.numpy as jnp
from jax import lax
from jax.experimental import pallas as pl
from jax.experimental.pallas import tpu as pltpu
from jax.experimental.pallas import tpu_sc as plsc

N = 10000
E = 320000
H = 128
L = 4

NC = 2
NS = 16
NW = NC * NS
CH = 128
K = -(-E // (NW * CH))
E_PAD = NW * K * CH
N_PAD = 10240
RPT = N_PAD // NS

_mesh = plsc.VectorSubcoreMesh(core_axis_name="c", subcore_axis_name="s")


def _zero_rows(buf, nrow):
    def row(i, _):
        for jj in range(H // 16):
            buf[i, pl.ds(jj * 16, 16)] = jnp.zeros((16,), jnp.float32)
        return 0
    lax.fori_loop(0, nrow, row, 0)


def _make_seg_sum(with_deg):
    out_type = [jax.ShapeDtypeStruct((NC, N_PAD, H), jnp.float32)]
    scratch = [
        pltpu.VMEM((K, CH), jnp.int32),
        pltpu.VMEM((K, CH), jnp.int32),
        pltpu.VMEM((CH, H), jnp.float32),
        pltpu.VMEM_SHARED((N_PAD, H), jnp.float32),
        pltpu.SemaphoreType.DMA,
    ]
    if with_deg:
        out_type.append(jax.ShapeDtypeStruct((NC, N_PAD), jnp.float32))
        scratch += [
            pltpu.VMEM((CH,), jnp.float32),
            pltpu.VMEM((RPT,), jnp.float32),
            pltpu.VMEM_SHARED((N_PAD,), jnp.float32),
        ]

    def body(*refs):
        if with_deg:
            (hl, srcr, dstr, sums, deg, sidx, didx, rows_a,
             acc, sem_a, ones_v, zvec, dacc) = refs
        else:
            (hl, srcr, dstr, sums, sidx, didx, rows_a,
             acc, sem_a) = refs
        c = lax.axis_index("c")
        s = lax.axis_index("s")
        wid = s * NC + c

        pltpu.sync_copy(srcr.at[wid], sidx)
        pltpu.sync_copy(dstr.at[wid], didx)

        _zero_rows(rows_a, CH)
        for t in range(RPT // CH):
            pltpu.sync_copy(rows_a, acc.at[pl.ds(s * RPT + t * CH, CH)])
        if with_deg:
            def fill(i, _):
                ones_v[pl.ds(i * 16, 16)] = jnp.ones((16,), jnp.float32)
                return 0
            lax.fori_loop(0, CH // 16, fill, 0)
            def zfill(i, _):
                zvec[pl.ds(i * 16, 16)] = jnp.zeros((16,), jnp.float32)
                return 0
            lax.fori_loop(0, RPT // 16, zfill, 0)
            pltpu.sync_copy(zvec, dacc.at[pl.ds(s * RPT, RPT)])
        plsc.subcore_barrier()

        def chunk(j, _):
            pltpu.async_copy(hl.at[sidx.at[j]], rows_a, sem_a).wait()
            pltpu.sync_copy(rows_a, acc.at[didx.at[j]], add=True)
            if with_deg:
                pltpu.sync_copy(ones_v, dacc.at[didx.at[j]], add=True)
            return 0

        lax.fori_loop(0, K, chunk, 0)
        plsc.subcore_barrier()

        for t in range(RPT // CH):
            off = s * RPT + t * CH
            pltpu.sync_copy(acc.at[pl.ds(off, CH)], sums.at[c, pl.ds(off, CH)])
        if with_deg:
            pltpu.sync_copy(dacc.at[pl.ds(s * RPT, RPT)],
                            deg.at[c, pl.ds(s * RPT, RPT)])

    return pl.kernel(body, out_type=out_type, mesh=_mesh,
                     scratch_types=scratch)


_seg_sum_deg = _make_seg_sum(True)
_seg_sum = _make_seg_sum(False)


BR = 1000
GRID = N // BR

_f32 = jnp.float32


def _dot(a, b):
    return jnp.dot(a, b, preferred_element_type=_f32)


def _entry_body(x, w_in, b_in, wl, wr, bl, h_o, hl_o, hr_o):
    h = _dot(x[...], w_in[...]) + b_in[...]
    h_o[...] = h
    hl_o[...] = _dot(h, wl[...])
    hr_o[...] = _dot(h, wr[...]) + bl[...]


def _mix(sums, deg, hr):
    total = sums[0] + sums[1]
    d = jnp.maximum(deg[:, 0:1] + deg[:, 1:2], 1.0)
    out = total / d + hr
    nrm = jnp.sqrt(jnp.sum(out * out, axis=-1, keepdims=True))
    out = out / jnp.maximum(nrm, 1e-12)
    mu = jnp.mean(out, axis=-1, keepdims=True)
    var = jnp.mean((out - mu) * (out - mu), axis=-1, keepdims=True)
    return (out - mu) / jnp.sqrt(var + 1e-5)


def _layer_body(sums, deg, hr, h_in, g, b, wl, wr, bl,
                h_o, hl_o, hr_o):
    out = _mix(sums[...], deg[...], hr[...]) * g[...] + b[...]
    out = out * jax.nn.sigmoid(out)
    h = out + h_in[...]
    h_o[...] = h
    hl_o[...] = _dot(h, wl[...])
    hr_o[...] = _dot(h, wr[...]) + bl[...]


def _final_body(sums, deg, hr, h_in, g, b, w1, b1, w2, b2, y_o):
    out = _mix(sums[...], deg[...], hr[...]) * g[...] + b[...]
    out = out * jax.nn.sigmoid(out)
    h = out + h_in[...]
    t = _dot(h, w1[...]) + b1[...]
    t = t * jax.nn.sigmoid(t)
    y_o[...] = _dot(t, w2[...]) + b2[...]


def _row_spec():
    return pl.BlockSpec((BR, H), lambda i: (i, 0))


def _w_spec():
    return pl.BlockSpec((H, H), lambda i: (0, 0))


def _b_spec():
    return pl.BlockSpec((1, H), lambda i: (0, 0))


def _sums_spec():
    return pl.BlockSpec((NC, BR, H), lambda i: (0, i, 0))


def _deg_spec():
    return pl.BlockSpec((BR, NC), lambda i: (i, 0))


_out3 = [jax.ShapeDtypeStruct((N, H), _f32)] * 3

_entry_call = pl.pallas_call(
    _entry_body,
    grid=(GRID,),
    in_specs=[_row_spec(), _w_spec(), _b_spec(), _w_spec(), _w_spec(),
              _b_spec()],
    out_specs=[_row_spec(), _row_spec(), _row_spec()],
    out_shape=_out3,
)

_layer_call = pl.pallas_call(
    _layer_body,
    grid=(GRID,),
    in_specs=[_sums_spec(), _deg_spec(), _row_spec(), _row_spec(),
              _b_spec(), _b_spec(), _w_spec(), _w_spec(), _b_spec()],
    out_specs=[_row_spec(), _row_spec(), _row_spec()],
    out_shape=_out3,
)

_final_call = pl.pallas_call(
    _final_body,
    grid=(GRID,),
    in_specs=[_sums_spec(), _deg_spec(), _row_spec(), _row_spec(),
              _b_spec(), _b_spec(), _w_spec(), _b_spec(), _w_spec(),
              _b_spec()],
    out_specs=_row_spec(),
    out_shape=jax.ShapeDtypeStruct((N, H), _f32),
)


def kernel(x, edge_index, W_in, b_in, Wl, bl, Wr, gamma, beta, W1, b1, W2, b2):
    src = edge_index[0].astype(jnp.int32)
    dst = edge_index[1].astype(jnp.int32)
    pad = E_PAD - E
    src_r = jnp.concatenate([src, jnp.zeros((pad,), jnp.int32)]).reshape(
        NW, K, CH)
    dst_r = jnp.concatenate([dst, jnp.full((pad,), N, jnp.int32)]).reshape(
        NW, K, CH)

    b_in2 = b_in.reshape(1, H)
    bl2 = bl.reshape(L, 1, H)
    g2 = gamma.reshape(L, 1, H)
    be2 = beta.reshape(L, 1, H)
    b12 = b1.reshape(1, H)
    W2p = jnp.zeros((H, H), _f32).at[:, :3].set(W2)
    b2p = jnp.zeros((1, H), _f32).at[0, :3].set(b2)

    h, hl, hr = _entry_call(x, W_in, b_in2, Wl[0], Wr[0], bl2[0])

    sums, deg = _seg_sum_deg(hl, src_r, dst_r)
    deg_t = jnp.swapaxes(deg, 0, 1)

    for i in range(L - 1):
        h, hl, hr = _layer_call(sums, deg_t, hr, h, g2[i], be2[i],
                                Wl[i + 1], Wr[i + 1], bl2[i + 1])
        (sums,) = _seg_sum(hl, src_r, dst_r)

    y = _final_call(sums, deg_t, hr, h, g2[L - 1], be2[L - 1],
                    W1, b12, W2p, b2p)
    return y[:, :3]
